# baseline (speedup 1.0000x reference)
import functools

import jax
import jax.numpy as jnp
from jax import lax
from jax.experimental import pallas as pl
from jax.experimental.pallas import tpu as pltpu
from jax.experimental.pallas import tpu_sc as plsc

B = 16384
F = 26
V = 100000
D = 32

VQ = 25600
VROW = 4 * VQ
NROWS = F * VROW

_info = plsc.get_sparse_core_info()
NC = _info.num_cores
NS = _info.num_subcores
LANES = _info.num_lanes
NW = NC * NS
ROWS_PER_W = B // NW
CB = 64
NCHUNK = ROWS_PER_W // CB
NIDX = CB * F


_TPC = 12800


def _tp_body(x0, x1, x2, x3, out_ref):
    eye = jnp.eye(D, dtype=jnp.float32)
    for s, x in enumerate((x0, x1, x2, x3)):
        out_ref[:, s * D:(s + 1) * D] = jax.lax.dot_general(
            x[0], eye, (((0,), (0,)), ((), ())),
            precision=jax.lax.Precision.HIGHEST)


def _tp_spec(s):
    return pl.BlockSpec((1, D, _TPC), lambda f, t: (f, 0, (VQ // _TPC) * s + t))


_tp_call = pl.pallas_call(
    _tp_body,
    grid=(F, VQ // _TPC),
    compiler_params=pltpu.CompilerParams(vmem_limit_bytes=100 * 1024 * 1024),
    in_specs=[_tp_spec(0), _tp_spec(1), _tp_spec(2), _tp_spec(3)],
    out_specs=pl.BlockSpec((_TPC, 4 * D),
                           lambda f, t: (f * (VQ // _TPC) + t, 0)),
    out_shape=jax.ShapeDtypeStruct((F * VQ, 4 * D), jnp.float32),
)


def _fm_body(eidx_hbm, lidx_hbm, emb_hbm, lin_hbm, p_hbm, eidx_v, lidx_v,
             rows_v, lin_v, p_v, esem0, esem1, lsem0, lsem1):
    wid = lax.axis_index("s") * NC + lax.axis_index("c")
    lane = lax.iota(jnp.int32, LANES)
    tail_mask = lane < (F - LANES)
    esems = (esem0, esem1)
    lsems = (lsem0, lsem1)

    def fire(c):
        slot = c % 2
        base = wid * ROWS_PER_W + c * CB
        ibase = pl.multiple_of(base * F, 8)
        eslice = eidx_v.at[pl.ds(slot * NIDX, NIDX)]
        lslice = lidx_v.at[pl.ds(slot * NIDX, NIDX)]
        pltpu.sync_copy(eidx_hbm.at[pl.ds(ibase, NIDX)], eslice)
        pltpu.sync_copy(lidx_hbm.at[pl.ds(ibase, NIDX)], lslice)
        cp_e = pltpu.async_copy(
            emb_hbm.at[eslice], rows_v.at[pl.ds(slot * NIDX, NIDX)],
            esems[slot])
        cp_l = pltpu.async_copy(
            lin_hbm.at[lslice],
            lin_v.at[pl.ds(slot * (NIDX + LANES), NIDX)], lsems[slot])
        return cp_e, cp_l

    def compute(c, cps):
        slot = c % 2
        base = wid * ROWS_PER_W + c * CB
        cps[0].wait()
        cps[1].wait()
        roff = slot * NIDX
        loff = slot * (NIDX + LANES)

        def row_body(i, _):
            rb = i * F
            zero = jnp.zeros((LANES,), jnp.float32)
            s0 = zero
            s1 = zero
            q = zero
            for f in range(F):
                e0 = rows_v[roff + rb + f, pl.ds(0, LANES)]
                e1 = rows_v[roff + rb + f, pl.ds(LANES, LANES)]
                s0 = s0 + e0
                s1 = s1 + e1
                q = q + (e0 * e0 + e1 * e1)
            r = s0 * s0 + s1 * s1 - q
            l0 = lin_v[pl.ds(loff + rb, LANES)]
            l1 = lin_v[pl.ds(loff + rb + LANES, LANES)]
            l1 = jnp.where(tail_mask, l1, 0.0)
            p_v[pl.ds(i * LANES, LANES)] = 0.5 * r + l0 + l1
            return 0

        lax.fori_loop(0, CB, row_body, 0, unroll=False)
        pltpu.sync_copy(
            p_v, p_hbm.at[pl.ds(pl.multiple_of(base * LANES, 8), CB * LANES)])

    cps = fire(0)
    for c in range(NCHUNK):
        nxt = fire(c + 1) if c + 1 < NCHUNK else None
        compute(c, cps)
        cps = nxt


_mesh = plsc.VectorSubcoreMesh(core_axis_name="c", subcore_axis_name="s")

_fm_call = functools.partial(
    pl.kernel,
    mesh=_mesh,
    compiler_params=pltpu.CompilerParams(use_tc_tiling_on_sc=False),
    out_type=jax.ShapeDtypeStruct((B * LANES,), jnp.float32),
    scratch_types=[
        pltpu.VMEM((2 * NIDX,), jnp.int32),
        pltpu.VMEM((2 * NIDX,), jnp.int32),
        pltpu.VMEM((2 * NIDX, D), jnp.float32),
        pltpu.VMEM((2 * (NIDX + LANES),), jnp.float32),
        pltpu.VMEM((CB * LANES,), jnp.float32),
        pltpu.SemaphoreType.DMA,
        pltpu.SemaphoreType.DMA,
        pltpu.SemaphoreType.DMA,
        pltpu.SemaphoreType.DMA,
    ],
)(_fm_body)


_TC_ROWS = 2048


def _reduce_body(bias_ref, p_ref, o_ref):
    o_ref[...] = jnp.sum(p_ref[...], axis=1) + bias_ref[0]


_reduce_call = pl.pallas_call(
    _reduce_body,
    grid=(B // _TC_ROWS,),
    in_specs=[
        pl.BlockSpec(memory_space=pltpu.SMEM),
        pl.BlockSpec((_TC_ROWS, LANES), lambda i: (i, 0)),
    ],
    out_specs=pl.BlockSpec((_TC_ROWS,), lambda i: (i,)),
    out_shape=jax.ShapeDtypeStruct((B,), jnp.float32),
)


def kernel(f0, f1, f2, f3, f4, f5, f6, f7, f8, f9, f10, f11, f12, f13, f14,
           f15, f16, f17, f18, f19, f20, f21, f22, f23, f24, f25, emb, lin,
           bias):
    feats = (f0, f1, f2, f3, f4, f5, f6, f7, f8, f9, f10, f11, f12, f13, f14,
             f15, f16, f17, f18, f19, f20, f21, f22, f23, f24, f25)
    idx = jnp.concatenate(feats, axis=1).astype(jnp.int32)
    foff = jnp.arange(F, dtype=jnp.int32)
    s = idx // VQ
    eidx = (foff * VROW + (idx - s * VQ) * 4 + s).reshape(-1)
    lidx = (idx + foff * V).reshape(-1)
    table = _tp_call(*((jnp.swapaxes(emb, 1, 2),) * 4)).reshape(NROWS, D)
    lin2 = lin.reshape(F * V)
    partials = _fm_call(eidx, lidx, table, lin2).reshape(B, LANES)
    return _reduce_call(bias, partials)


# speedup vs baseline: 3.4177x; 3.4177x over previous
import functools

import jax
import jax.numpy as jnp
from jax import lax
from jax.experimental import pallas as pl
from jax.experimental.pallas import tpu as pltpu
from jax.experimental.pallas import tpu_sc as plsc

B = 16384
F = 26
V = 100000
D = 32

VQ = 25600
VROW = 4 * VQ
NROWS = F * VROW

_info = plsc.get_sparse_core_info()
NC = _info.num_cores
NS = _info.num_subcores
LANES = _info.num_lanes
NW = NC * NS
ROWS_PER_W = B // NW
CB = 64
NCHUNK = ROWS_PER_W // CB
NIDX = CB * F


_TPC = 12800


def _tp_body(x0, x1, x2, x3, out_ref):
    w = jnp.concatenate([x0[0], x1[0], x2[0], x3[0]], axis=0)
    eye = jnp.eye(4 * D, dtype=jnp.float32)
    out_ref[...] = jax.lax.dot_general(
        w, eye, (((0,), (0,)), ((), ())),
        precision=jax.lax.Precision.HIGHEST)


def _tp_spec(s):
    return pl.BlockSpec((1, D, _TPC), lambda f, t: (f, 0, (VQ // _TPC) * s + t))


_tp_call = pl.pallas_call(
    _tp_body,
    grid=(F, VQ // _TPC),
    compiler_params=pltpu.CompilerParams(vmem_limit_bytes=100 * 1024 * 1024),
    in_specs=[_tp_spec(0), _tp_spec(1), _tp_spec(2), _tp_spec(3)],
    out_specs=pl.BlockSpec((_TPC, 4 * D),
                           lambda f, t: (f * (VQ // _TPC) + t, 0)),
    out_shape=jax.ShapeDtypeStruct((F * VQ, 4 * D), jnp.float32),
)


def _fm_body(eidx_hbm, lidx_hbm, emb_hbm, lin_hbm, p_hbm, eidx_v, lidx_v,
             rows_v, lin_v, p_v, esem0, esem1, lsem0, lsem1):
    wid = lax.axis_index("s") * NC + lax.axis_index("c")
    lane = lax.iota(jnp.int32, LANES)
    tail_mask = lane < (F - LANES)
    esems = (esem0, esem1)
    lsems = (lsem0, lsem1)

    def fire(c):
        slot = c % 2
        base = wid * ROWS_PER_W + c * CB
        ibase = pl.multiple_of(base * F, 8)
        eslice = eidx_v.at[pl.ds(slot * NIDX, NIDX)]
        lslice = lidx_v.at[pl.ds(slot * NIDX, NIDX)]
        pltpu.sync_copy(eidx_hbm.at[pl.ds(ibase, NIDX)], eslice)
        pltpu.sync_copy(lidx_hbm.at[pl.ds(ibase, NIDX)], lslice)
        cp_e = pltpu.async_copy(
            emb_hbm.at[eslice], rows_v.at[pl.ds(slot * NIDX, NIDX)],
            esems[slot])
        cp_l = pltpu.async_copy(
            lin_hbm.at[lslice],
            lin_v.at[pl.ds(slot * (NIDX + LANES), NIDX)], lsems[slot])
        return cp_e, cp_l

    def compute(c, cps):
        slot = c % 2
        base = wid * ROWS_PER_W + c * CB
        cps[0].wait()
        cps[1].wait()
        roff = slot * NIDX
        loff = slot * (NIDX + LANES)

        def row_body(i, _):
            rb = i * F
            zero = jnp.zeros((LANES,), jnp.float32)
            s0 = zero
            s1 = zero
            q = zero
            for f in range(F):
                e0 = rows_v[roff + rb + f, pl.ds(0, LANES)]
                e1 = rows_v[roff + rb + f, pl.ds(LANES, LANES)]
                s0 = s0 + e0
                s1 = s1 + e1
                q = q + (e0 * e0 + e1 * e1)
            r = s0 * s0 + s1 * s1 - q
            l0 = lin_v[pl.ds(loff + rb, LANES)]
            l1 = lin_v[pl.ds(loff + rb + LANES, LANES)]
            l1 = jnp.where(tail_mask, l1, 0.0)
            p_v[pl.ds(i * LANES, LANES)] = 0.5 * r + l0 + l1
            return 0

        lax.fori_loop(0, CB, row_body, 0, unroll=False)
        pltpu.sync_copy(
            p_v, p_hbm.at[pl.ds(pl.multiple_of(base * LANES, 8), CB * LANES)])

    cps = fire(0)
    for c in range(NCHUNK):
        nxt = fire(c + 1) if c + 1 < NCHUNK else None
        compute(c, cps)
        cps = nxt


_mesh = plsc.VectorSubcoreMesh(core_axis_name="c", subcore_axis_name="s")

_fm_call = functools.partial(
    pl.kernel,
    mesh=_mesh,
    compiler_params=pltpu.CompilerParams(use_tc_tiling_on_sc=False),
    out_type=jax.ShapeDtypeStruct((B * LANES,), jnp.float32),
    scratch_types=[
        pltpu.VMEM((2 * NIDX,), jnp.int32),
        pltpu.VMEM((2 * NIDX,), jnp.int32),
        pltpu.VMEM((2 * NIDX, D), jnp.float32),
        pltpu.VMEM((2 * (NIDX + LANES),), jnp.float32),
        pltpu.VMEM((CB * LANES,), jnp.float32),
        pltpu.SemaphoreType.DMA,
        pltpu.SemaphoreType.DMA,
        pltpu.SemaphoreType.DMA,
        pltpu.SemaphoreType.DMA,
    ],
)(_fm_body)


_TC_ROWS = 2048


def _reduce_body(bias_ref, p_ref, o_ref):
    o_ref[...] = jnp.sum(p_ref[...], axis=1) + bias_ref[0]


_reduce_call = pl.pallas_call(
    _reduce_body,
    grid=(B // _TC_ROWS,),
    in_specs=[
        pl.BlockSpec(memory_space=pltpu.SMEM),
        pl.BlockSpec((_TC_ROWS, LANES), lambda i: (i, 0)),
    ],
    out_specs=pl.BlockSpec((_TC_ROWS,), lambda i: (i,)),
    out_shape=jax.ShapeDtypeStruct((B,), jnp.float32),
)


def kernel(f0, f1, f2, f3, f4, f5, f6, f7, f8, f9, f10, f11, f12, f13, f14,
           f15, f16, f17, f18, f19, f20, f21, f22, f23, f24, f25, emb, lin,
           bias):
    feats = (f0, f1, f2, f3, f4, f5, f6, f7, f8, f9, f10, f11, f12, f13, f14,
             f15, f16, f17, f18, f19, f20, f21, f22, f23, f24, f25)
    idx = jnp.concatenate(feats, axis=1).astype(jnp.int32)
    foff = jnp.arange(F, dtype=jnp.int32)
    s = idx // VQ
    eidx = (foff * VROW + (idx - s * VQ) * 4 + s).reshape(-1)
    lidx = (idx + foff * V).reshape(-1)
    table = _tp_call(*((jnp.swapaxes(emb, 1, 2),) * 4)).reshape(NROWS, D)
    lin2 = lin.reshape(F * V)
    partials = _fm_call(eidx, lidx, table, lin2).reshape(B, LANES)
    return _reduce_call(bias, partials)


# speedup vs baseline: 4.0722x; 1.1915x over previous
import functools

import jax
import jax.numpy as jnp
from jax import lax
from jax.experimental import pallas as pl
from jax.experimental.pallas import tpu as pltpu
from jax.experimental.pallas import tpu_sc as plsc

B = 16384
F = 26
V = 100000
D = 32

VQ = 25600
VROW = 4 * VQ
NROWS = F * VROW

_info = plsc.get_sparse_core_info()
NC = _info.num_cores
NS = _info.num_subcores
LANES = _info.num_lanes
NW = NC * NS
ROWS_PER_W = B // NW
CB = 64
NCHUNK = ROWS_PER_W // CB
NIDX = CB * F


_TPC = 12800


def _tp_body(x0, x1, x2, x3, out_ref):
    w = jnp.concatenate([x0[0], x1[0], x2[0], x3[0]], axis=0)
    eye = jnp.eye(4 * D, dtype=jnp.bfloat16)
    hi = w.astype(jnp.bfloat16)
    r1 = w - hi.astype(jnp.float32)
    mid = r1.astype(jnp.bfloat16)
    lo = (r1 - mid.astype(jnp.float32)).astype(jnp.bfloat16)
    dn = (((0,), (0,)), ((), ()))

    def t(p):
        return jax.lax.dot_general(p, eye, dn,
                                   preferred_element_type=jnp.float32)

    out_ref[...] = t(hi) + t(mid) + t(lo)


def _tp_spec(s):
    return pl.BlockSpec((1, D, _TPC), lambda f, t: (f, 0, (VQ // _TPC) * s + t))


_tp_call = pl.pallas_call(
    _tp_body,
    grid=(F, VQ // _TPC),
    compiler_params=pltpu.CompilerParams(vmem_limit_bytes=100 * 1024 * 1024),
    in_specs=[_tp_spec(0), _tp_spec(1), _tp_spec(2), _tp_spec(3)],
    out_specs=pl.BlockSpec((_TPC, 4 * D),
                           lambda f, t: (f * (VQ // _TPC) + t, 0)),
    out_shape=jax.ShapeDtypeStruct((F * VQ, 4 * D), jnp.float32),
)


def _fm_body(eidx_hbm, lidx_hbm, emb_hbm, lin_hbm, p_hbm, eidx_v, lidx_v,
             rows_v, lin_v, p_v, esem0, esem1, lsem0, lsem1):
    wid = lax.axis_index("s") * NC + lax.axis_index("c")
    lane = lax.iota(jnp.int32, LANES)
    tail_mask = lane < (F - LANES)
    esems = (esem0, esem1)
    lsems = (lsem0, lsem1)

    def fire(c):
        slot = c % 2
        base = wid * ROWS_PER_W + c * CB
        ibase = pl.multiple_of(base * F, 8)
        eslice = eidx_v.at[pl.ds(slot * NIDX, NIDX)]
        lslice = lidx_v.at[pl.ds(slot * NIDX, NIDX)]
        pltpu.sync_copy(eidx_hbm.at[pl.ds(ibase, NIDX)], eslice)
        pltpu.sync_copy(lidx_hbm.at[pl.ds(ibase, NIDX)], lslice)
        cp_e = pltpu.async_copy(
            emb_hbm.at[eslice], rows_v.at[pl.ds(slot * NIDX, NIDX)],
            esems[slot])
        cp_l = pltpu.async_copy(
            lin_hbm.at[lslice],
            lin_v.at[pl.ds(slot * (NIDX + LANES), NIDX)], lsems[slot])
        return cp_e, cp_l

    def compute(c, cps):
        slot = c % 2
        base = wid * ROWS_PER_W + c * CB
        cps[0].wait()
        cps[1].wait()
        roff = slot * NIDX
        loff = slot * (NIDX + LANES)

        def row_body(i, _):
            rb = i * F
            zero = jnp.zeros((LANES,), jnp.float32)
            s0 = zero
            s1 = zero
            q = zero
            for f in range(F):
                e0 = rows_v[roff + rb + f, pl.ds(0, LANES)]
                e1 = rows_v[roff + rb + f, pl.ds(LANES, LANES)]
                s0 = s0 + e0
                s1 = s1 + e1
                q = q + (e0 * e0 + e1 * e1)
            r = s0 * s0 + s1 * s1 - q
            l0 = lin_v[pl.ds(loff + rb, LANES)]
            l1 = lin_v[pl.ds(loff + rb + LANES, LANES)]
            l1 = jnp.where(tail_mask, l1, 0.0)
            p_v[pl.ds(i * LANES, LANES)] = 0.5 * r + l0 + l1
            return 0

        lax.fori_loop(0, CB, row_body, 0, unroll=False)
        pltpu.sync_copy(
            p_v, p_hbm.at[pl.ds(pl.multiple_of(base * LANES, 8), CB * LANES)])

    cps = fire(0)
    for c in range(NCHUNK):
        nxt = fire(c + 1) if c + 1 < NCHUNK else None
        compute(c, cps)
        cps = nxt


_mesh = plsc.VectorSubcoreMesh(core_axis_name="c", subcore_axis_name="s")

_fm_call = functools.partial(
    pl.kernel,
    mesh=_mesh,
    compiler_params=pltpu.CompilerParams(use_tc_tiling_on_sc=False),
    out_type=jax.ShapeDtypeStruct((B * LANES,), jnp.float32),
    scratch_types=[
        pltpu.VMEM((2 * NIDX,), jnp.int32),
        pltpu.VMEM((2 * NIDX,), jnp.int32),
        pltpu.VMEM((2 * NIDX, D), jnp.float32),
        pltpu.VMEM((2 * (NIDX + LANES),), jnp.float32),
        pltpu.VMEM((CB * LANES,), jnp.float32),
        pltpu.SemaphoreType.DMA,
        pltpu.SemaphoreType.DMA,
        pltpu.SemaphoreType.DMA,
        pltpu.SemaphoreType.DMA,
    ],
)(_fm_body)


_TC_ROWS = 2048


def _reduce_body(bias_ref, p_ref, o_ref):
    o_ref[...] = jnp.sum(p_ref[...], axis=1) + bias_ref[0]


_reduce_call = pl.pallas_call(
    _reduce_body,
    grid=(B // _TC_ROWS,),
    in_specs=[
        pl.BlockSpec(memory_space=pltpu.SMEM),
        pl.BlockSpec((_TC_ROWS, LANES), lambda i: (i, 0)),
    ],
    out_specs=pl.BlockSpec((_TC_ROWS,), lambda i: (i,)),
    out_shape=jax.ShapeDtypeStruct((B,), jnp.float32),
)


def kernel(f0, f1, f2, f3, f4, f5, f6, f7, f8, f9, f10, f11, f12, f13, f14,
           f15, f16, f17, f18, f19, f20, f21, f22, f23, f24, f25, emb, lin,
           bias):
    feats = (f0, f1, f2, f3, f4, f5, f6, f7, f8, f9, f10, f11, f12, f13, f14,
             f15, f16, f17, f18, f19, f20, f21, f22, f23, f24, f25)
    idx = jnp.concatenate(feats, axis=1).astype(jnp.int32)
    foff = jnp.arange(F, dtype=jnp.int32)
    s = idx // VQ
    eidx = (foff * VROW + (idx - s * VQ) * 4 + s).reshape(-1)
    lidx = (idx + foff * V).reshape(-1)
    table = _tp_call(*((jnp.swapaxes(emb, 1, 2),) * 4)).reshape(NROWS, D)
    lin2 = lin.reshape(F * V)
    partials = _fm_call(eidx, lidx, table, lin2).reshape(B, LANES)
    return _reduce_call(bias, partials)
